# baseline (speedup 1.0000x reference)
import jax
import jax.numpy as jnp
from jax import lax
from jax.experimental import pallas as pl
from jax.experimental.pallas import tpu as pltpu
from jax.experimental.pallas import tpu_sc as plsc

B, T, D = 32, 2047, 1024
NC, NS = 2, 16
CHUNK = 40
SC_ROWS = 2040
NFULL = SC_ROWS // CHUNK


def _sc_body(cache_hbm, s_hbm, order_hbm, out_hbm, ord_v, sem):
    wid = lax.axis_index("s") * NC + lax.axis_index("c")

    pltpu.sync_copy(order_hbm, ord_v)
    lanes = lax.iota(jnp.int32, 16)
    zero = jnp.zeros((16,), jnp.int32)
    va = jnp.where(lanes == wid, ord_v[pl.ds(0, 16)], zero)
    vb = jnp.where(lanes + 16 == wid, ord_v[pl.ds(16, 16)], zero)
    src = jnp.max(va + vb)

    pltpu.async_copy(
        cache_hbm.at[src, pl.ds(0, SC_ROWS)], out_hbm.at[wid, pl.ds(0, SC_ROWS)], sem
    ).wait()


def _sc_bulk(cache, s, order):
    mesh = plsc.VectorSubcoreMesh(
        core_axis_name="c", subcore_axis_name="s", num_cores=NC, num_subcores=NS
    )
    return pl.kernel(
        _sc_body,
        out_type=jax.ShapeDtypeStruct((B, T + 1, D), jnp.float32),
        mesh=mesh,
        compiler_params=pltpu.CompilerParams(needs_layout_passes=False),
        scratch_types=[
            pltpu.VMEM((B,), jnp.int32),
            pltpu.SemaphoreType.DMA,
        ],
    )(cache, s, order)


def _tc_tail_body(order_ref, cache_ref, s_ref, prev_ref, out_ref):
    del order_ref, prev_ref
    blk = cache_ref[0]
    out_ref[0, :7] = blk[:7]
    out_ref[0, 7:8] = s_ref[0]


def _tc_tail(cache, s, order, prev):
    grid_spec = pltpu.PrefetchScalarGridSpec(
        num_scalar_prefetch=1,
        grid=(B,),
        in_specs=[
            pl.BlockSpec((1, 8, D), lambda i, ord_ref: (ord_ref[i], T // 8, 0)),
            pl.BlockSpec((1, 1, D), lambda i, ord_ref: (ord_ref[i], 0, 0)),
            pl.BlockSpec(memory_space=pl.ANY),
        ],
        out_specs=pl.BlockSpec((1, 8, D), lambda i, ord_ref: (i, T // 8, 0)),
    )
    return pl.pallas_call(
        _tc_tail_body,
        grid_spec=grid_spec,
        out_shape=jax.ShapeDtypeStruct((B, T + 1, D), jnp.float32),
        input_output_aliases={3: 0},
    )(order, cache, s, prev)


@jax.jit
def kernel(cache, s, order):
    return _tc_tail(cache, s, order, _sc_bulk(cache, s, order))


# speedup vs baseline: 19.2090x; 19.2090x over previous
import jax
import jax.numpy as jnp
from jax import lax
from jax.experimental import pallas as pl
from jax.experimental.pallas import tpu as pltpu
from jax.experimental.pallas import tpu_sc as plsc

B, T, D = 32, 2047, 1024
NC, NS = 2, 16
CHUNK = 40
NBUF = 3
SC_ROWS = 2040
NFULL = SC_ROWS // CHUNK


def _sc_body(cache_hbm, s_hbm, order_hbm, out_hbm, ord_v, bufs, gsems, osems):
    wid = lax.axis_index("s") * NC + lax.axis_index("c")

    pltpu.sync_copy(order_hbm, ord_v)
    lanes = lax.iota(jnp.int32, 16)
    zero = jnp.zeros((16,), jnp.int32)
    va = jnp.where(lanes == wid, ord_v[pl.ds(0, 16)], zero)
    vb = jnp.where(lanes + 16 == wid, ord_v[pl.ds(16, 16)], zero)
    src = jnp.max(va + vb)

    gathers = [None] * NFULL
    stores = [None] * NFULL

    def gather(c):
        return pltpu.async_copy(
            cache_hbm.at[src, pl.ds(c * CHUNK, CHUNK)], bufs[c % NBUF], gsems[c % NBUF]
        )

    def store(c):
        return pltpu.async_copy(
            bufs[c % NBUF], out_hbm.at[wid, pl.ds(c * CHUNK, CHUNK)], osems[c % NBUF]
        )

    for c in range(NBUF - 1):
        gathers[c] = gather(c)
    for c in range(NFULL):
        n = c + NBUF - 1
        if n < NFULL:
            if c >= 1:
                stores[c - 1].wait()
            gathers[n] = gather(n)
        gathers[c].wait()
        stores[c] = store(c)
    for c in range(NFULL - NBUF, NFULL):
        stores[c].wait()


def _sc_bulk(cache, s, order):
    mesh = plsc.VectorSubcoreMesh(
        core_axis_name="c", subcore_axis_name="s", num_cores=NC, num_subcores=NS
    )
    return pl.kernel(
        _sc_body,
        out_type=jax.ShapeDtypeStruct((B, T + 1, D), jnp.float32),
        mesh=mesh,
        compiler_params=pltpu.CompilerParams(needs_layout_passes=False),
        scratch_types=[
            pltpu.VMEM((B,), jnp.int32),
            [pltpu.VMEM((CHUNK, D), jnp.float32) for _ in range(NBUF)],
            [pltpu.SemaphoreType.DMA for _ in range(NBUF)],
            [pltpu.SemaphoreType.DMA for _ in range(NBUF)],
        ],
    )(cache, s, order)


def _tc_tail_body(order_ref, cache_ref, s_ref, prev_ref, out_ref):
    del order_ref, prev_ref
    blk = cache_ref[0]
    out_ref[0, :7] = blk[:7]
    out_ref[0, 7:8] = s_ref[0]


def _tc_tail(cache, s, order, prev):
    grid_spec = pltpu.PrefetchScalarGridSpec(
        num_scalar_prefetch=1,
        grid=(B,),
        in_specs=[
            pl.BlockSpec((1, 8, D), lambda i, ord_ref: (ord_ref[i], T // 8, 0)),
            pl.BlockSpec((1, 1, D), lambda i, ord_ref: (ord_ref[i], 0, 0)),
            pl.BlockSpec(memory_space=pl.ANY),
        ],
        out_specs=pl.BlockSpec((1, 8, D), lambda i, ord_ref: (i, T // 8, 0)),
    )
    return pl.pallas_call(
        _tc_tail_body,
        grid_spec=grid_spec,
        out_shape=jax.ShapeDtypeStruct((B, T + 1, D), jnp.float32),
        input_output_aliases={3: 0},
    )(order, cache, s, prev)


@jax.jit
def kernel(cache, s, order):
    return _tc_tail(cache, s, order, _sc_bulk(cache, s, order))


# speedup vs baseline: 31.0278x; 1.6153x over previous
import jax
import jax.numpy as jnp
from jax import lax
from jax.experimental import pallas as pl
from jax.experimental.pallas import tpu as pltpu
from jax.experimental.pallas import tpu_sc as plsc

B, T, D = 32, 2047, 1024
NC, NS = 2, 16
NW = NC * NS
TT = 8
NG = B // 8
SC_ROWS = 2040
NTC = SC_ROWS // TT
NUNITS = NTC * NG
KMAX = (NUNITS + NW - 1) // NW


def _sc_body(cache_t_hbm, order_hbm, out_hbm, ord_v, slab, osem):
    wid = lax.axis_index("s") * NC + lax.axis_index("c")

    pltpu.sync_copy(order_hbm, ord_v)
    lanes = lax.iota(jnp.int32, 16)
    zero = jnp.zeros((16,), jnp.int32)
    lo = ord_v[pl.ds(0, 16)]
    hi = ord_v[pl.ds(16, 16)]
    srcs = [
        jnp.max(jnp.where(lanes == i, lo, zero)) if i < 16
        else jnp.max(jnp.where(lanes == i - 16, hi, zero))
        for i in range(B)
    ]

    def unit(k, carry):
        u = wid + NW * k
        valid = u < NUNITS
        g = u % NG
        tc = u // NG
        t0 = pl.multiple_of(tc * TT, TT)
        b0 = pl.multiple_of(g * 8, 8)

        conds = [valid & (srcs[i] // 8 == g) for i in range(B)]
        any_hit = conds[0]
        for i in range(1, B):
            any_hit = any_hit | conds[i]

        @pl.when(any_hit)
        def _():
            pltpu.sync_copy(cache_t_hbm.at[pl.ds(t0, TT), pl.ds(b0, 8)], slab)
            for i in range(B):
                @pl.when(conds[i])
                def _(i=i):
                    pltpu.make_async_copy(
                        slab.at[:, srcs[i] % 8, :],
                        out_hbm.at[i, pl.ds(t0, TT)],
                        osem,
                    ).start()
            for i in range(B):
                @pl.when(conds[i])
                def _(i=i):
                    pltpu.make_async_copy(
                        slab.at[:, srcs[i] % 8, :],
                        out_hbm.at[i, pl.ds(t0, TT)],
                        osem,
                    ).wait()

        return carry

    lax.fori_loop(0, KMAX, unit, 0)


def _sc_bulk(cache_t, order):
    mesh = plsc.VectorSubcoreMesh(
        core_axis_name="c", subcore_axis_name="s", num_cores=NC, num_subcores=NS
    )
    return pl.kernel(
        _sc_body,
        out_type=jax.ShapeDtypeStruct((B, T + 1, D), jnp.float32),
        mesh=mesh,
        compiler_params=pltpu.CompilerParams(needs_layout_passes=False),
        scratch_types=[
            pltpu.VMEM((B,), jnp.int32),
            pltpu.VMEM((TT, 8, D), jnp.float32),
            pltpu.SemaphoreType.DMA,
        ],
    )(cache_t, order)


def _tc_tail_body(order_ref, cache_t_ref, s_ref, prev_ref, out_ref):
    del prev_ref
    i = pl.program_id(0)
    src = order_ref[i]
    blk = cache_t_ref[:, pl.ds(src, 1), :]
    out_ref[0, :7] = blk[:7, 0, :]
    out_ref[0, 7:8] = s_ref[0]


def _tc_tail(cache_t, s, order, prev):
    grid_spec = pltpu.PrefetchScalarGridSpec(
        num_scalar_prefetch=1,
        grid=(B,),
        in_specs=[
            pl.BlockSpec((8, B, D), lambda i, ord_ref: (T // 8, 0, 0)),
            pl.BlockSpec((1, 1, D), lambda i, ord_ref: (ord_ref[i], 0, 0)),
            pl.BlockSpec(memory_space=pl.ANY),
        ],
        out_specs=pl.BlockSpec((1, 8, D), lambda i, ord_ref: (i, T // 8, 0)),
    )
    return pl.pallas_call(
        _tc_tail_body,
        grid_spec=grid_spec,
        out_shape=jax.ShapeDtypeStruct((B, T + 1, D), jnp.float32),
        input_output_aliases={3: 0},
    )(order, cache_t, s, prev)


@jax.jit
def kernel(cache, s, order):
    cache_t = jnp.transpose(cache, (1, 0, 2))
    return _tc_tail(cache_t, s, order, _sc_bulk(cache_t, order))


# speedup vs baseline: 33.5312x; 1.0807x over previous
import jax
import jax.numpy as jnp
from jax import lax
from jax.experimental import pallas as pl
from jax.experimental.pallas import tpu as pltpu
from jax.experimental.pallas import tpu_sc as plsc

B, T, D = 32, 2047, 1024
NC, NS = 2, 16
NW = NC * NS
TT = 8
NG = B // 8
SC_ROWS = 2040
NTC = SC_ROWS // TT
NUNITS = NTC * NG
KMAX = (NUNITS + NW - 1) // NW


def _sc_body(cache_t_hbm, order_hbm, out_hbm, ord_v, slab, gsem, osem):
    wid = lax.axis_index("s") * NC + lax.axis_index("c")

    pltpu.sync_copy(order_hbm, ord_v)
    lanes = lax.iota(jnp.int32, 16)
    zero = jnp.zeros((16,), jnp.int32)
    lo = ord_v[pl.ds(0, 16)]
    hi = ord_v[pl.ds(16, 16)]
    srcs = [
        jnp.max(jnp.where(lanes == i, lo, zero)) if i < 16
        else jnp.max(jnp.where(lanes == i - 16, hi, zero))
        for i in range(B)
    ]

    def unit(k, carry):
        u = wid + NW * k
        valid = u < NUNITS
        g = u % NG
        tc = u // NG
        t0 = pl.multiple_of(tc * TT, TT)
        b0 = pl.multiple_of(g * 8, 8)

        conds = [valid & (srcs[i] // 8 == g) for i in range(B)]
        any_hit = conds[0]
        for i in range(1, B):
            any_hit = any_hit | conds[i]

        @pl.when(any_hit)
        def _():
            for tt in range(TT):
                pltpu.make_async_copy(
                    cache_t_hbm.at[t0 + tt, pl.ds(b0, 8)],
                    slab.at[:, tt],
                    gsem,
                ).start()
            for tt in range(TT):
                pltpu.make_async_copy(
                    cache_t_hbm.at[t0 + tt, pl.ds(b0, 8)],
                    slab.at[:, tt],
                    gsem,
                ).wait()
            for i in range(B):
                @pl.when(conds[i])
                def _(i=i):
                    pltpu.make_async_copy(
                        slab.at[srcs[i] % 8],
                        out_hbm.at[i, pl.ds(t0, TT)],
                        osem,
                    ).start()
            for i in range(B):
                @pl.when(conds[i])
                def _(i=i):
                    pltpu.make_async_copy(
                        slab.at[srcs[i] % 8],
                        out_hbm.at[i, pl.ds(t0, TT)],
                        osem,
                    ).wait()

        return carry

    lax.fori_loop(0, KMAX, unit, 0)


def _sc_bulk(cache_t, order):
    mesh = plsc.VectorSubcoreMesh(
        core_axis_name="c", subcore_axis_name="s", num_cores=NC, num_subcores=NS
    )
    return pl.kernel(
        _sc_body,
        out_type=jax.ShapeDtypeStruct((B, T + 1, D), jnp.float32),
        mesh=mesh,
        compiler_params=pltpu.CompilerParams(needs_layout_passes=False),
        scratch_types=[
            pltpu.VMEM((B,), jnp.int32),
            pltpu.VMEM((8, TT, D), jnp.float32),
            pltpu.SemaphoreType.DMA,
            pltpu.SemaphoreType.DMA,
        ],
    )(cache_t, order)


def _tc_tail_body(order_ref, cache_t_ref, s_ref, prev_ref, out_ref):
    del prev_ref
    i = pl.program_id(0)
    src = order_ref[i]
    blk = cache_t_ref[:, pl.ds(src, 1), :]
    out_ref[0, :7] = blk[:7, 0, :]
    out_ref[0, 7:8] = s_ref[0]


def _tc_tail(cache_t, s, order, prev):
    grid_spec = pltpu.PrefetchScalarGridSpec(
        num_scalar_prefetch=1,
        grid=(B,),
        in_specs=[
            pl.BlockSpec((8, B, D), lambda i, ord_ref: (T // 8, 0, 0)),
            pl.BlockSpec((1, 1, D), lambda i, ord_ref: (ord_ref[i], 0, 0)),
            pl.BlockSpec(memory_space=pl.ANY),
        ],
        out_specs=pl.BlockSpec((1, 8, D), lambda i, ord_ref: (i, T // 8, 0)),
    )
    return pl.pallas_call(
        _tc_tail_body,
        grid_spec=grid_spec,
        out_shape=jax.ShapeDtypeStruct((B, T + 1, D), jnp.float32),
        input_output_aliases={3: 0},
    )(order, cache_t, s, prev)


@jax.jit
def kernel(cache, s, order):
    cache_t = jnp.transpose(cache, (1, 0, 2))
    return _tc_tail(cache_t, s, order, _sc_bulk(cache_t, order))
